# Initial kernel scaffold; baseline (speedup 1.0000x reference)
import jax
import jax.numpy as jnp
from jax.experimental import pallas as pl


def kernel(x, edge_index, batch, W_rel, b_rel, W_root, gamma, beta, lin_W, lin_b):
    raise NotImplementedError("write your pallas kernel here")


# speedup vs baseline: 4.5253x; 4.5253x over previous
import functools

import jax
import jax.numpy as jnp
from jax import lax
from jax.experimental import pallas as pl
from jax.experimental.pallas import tpu as pltpu
from jax.experimental.pallas import tpu_sc as plsc

_NC = 2
_NS = 16
_K = 80


@functools.cache
def _sc_aggregate(n, d, e):
    nw = _NC * _NS
    ew = e // nw
    steps = ew // _K
    rows_main = (n // _NS) & ~7
    tail_off = rows_main * _NS
    tail_len = n - tail_off
    mesh = plsc.VectorSubcoreMesh(core_axis_name="c", subcore_axis_name="s")

    @functools.partial(
        pl.kernel,
        mesh=mesh,
        out_type=jax.ShapeDtypeStruct((_NC, n, d), jnp.float32),
        scratch_types=[
            pltpu.VMEM_SHARED((n, d), jnp.float32),
            pltpu.VMEM((_K,), jnp.int32),
            pltpu.VMEM((_K,), jnp.int32),
            pltpu.VMEM((_K, d), jnp.float32),
            pltpu.SemaphoreType.DMA,
        ],
    )
    def agg(h_hbm, src_hbm, dst_hbm, zeros_hbm, out_hbm,
            aggr_sm, srcv, dstv, rows, sem):
        c = lax.axis_index("c")
        s = lax.axis_index("s")
        pltpu.sync_copy(
            zeros_hbm.at[pl.ds(s * rows_main, rows_main)],
            aggr_sm.at[pl.ds(s * rows_main, rows_main)],
        )
        if tail_len:
            @pl.when(s == _NS - 1)
            def _():
                pltpu.sync_copy(
                    zeros_hbm.at[pl.ds(tail_off, tail_len)],
                    aggr_sm.at[pl.ds(tail_off, tail_len)],
                )
        plsc.subcore_barrier()
        base = (c * _NS + s) * ew

        def body(i, carry):
            off = base + i * _K
            pltpu.sync_copy(src_hbm.at[pl.ds(off, _K)], srcv)
            pltpu.sync_copy(dst_hbm.at[pl.ds(off, _K)], dstv)
            pltpu.async_copy(h_hbm.at[srcv], rows, sem).wait()
            pltpu.sync_copy(rows, aggr_sm.at[dstv], add=True)
            return carry

        lax.fori_loop(0, steps, body, 0)
        plsc.subcore_barrier()
        pltpu.sync_copy(
            aggr_sm.at[pl.ds(s * rows_main, rows_main)],
            out_hbm.at[c].at[pl.ds(s * rows_main, rows_main)],
        )
        if tail_len:
            @pl.when(s == _NS - 1)
            def _():
                pltpu.sync_copy(
                    aggr_sm.at[pl.ds(tail_off, tail_len)],
                    out_hbm.at[c].at[pl.ds(tail_off, tail_len)],
                )

    return agg


def _tc_layer_body(p_ref, h_ref, wr_ref, br_ref, wt_ref, g_ref, b_ref, o_ref):
    aggr = p_ref[0] + p_ref[1]
    h = h_ref[...]
    y = (jnp.dot(aggr, wr_ref[...], preferred_element_type=jnp.float32)
         + jnp.dot(h, wt_ref[...], preferred_element_type=jnp.float32)
         + br_ref[...])
    m = jnp.mean(y, axis=0, keepdims=True)
    v = jnp.mean((y - m) ** 2, axis=0, keepdims=True)
    o_ref[...] = jnp.maximum(
        g_ref[...] * (y - m) * lax.rsqrt(v + 1e-5) + b_ref[...], 0.0)


@functools.cache
def _tc_layer(n, d):
    return pl.pallas_call(
        _tc_layer_body,
        out_shape=jax.ShapeDtypeStruct((n, d), jnp.float32),
    )


def _tc_final_body(p_ref, h_ref, wr_ref, br_ref, wt_ref, g_ref, b_ref,
                   batch_ref, lw_ref, lb_ref, o_ref, *, num_graphs):
    aggr = p_ref[0] + p_ref[1]
    h = h_ref[...]
    y = (jnp.dot(aggr, wr_ref[...], preferred_element_type=jnp.float32)
         + jnp.dot(h, wt_ref[...], preferred_element_type=jnp.float32)
         + br_ref[...])
    m = jnp.mean(y, axis=0, keepdims=True)
    v = jnp.mean((y - m) ** 2, axis=0, keepdims=True)
    hlast = jnp.maximum(
        g_ref[...] * (y - m) * lax.rsqrt(v + 1e-5) + b_ref[...], 0.0)
    n = hlast.shape[0]
    seg = lax.broadcasted_iota(jnp.int32, (n, num_graphs), 1)
    onehot = (batch_ref[...] == seg).astype(jnp.float32)
    sums = lax.dot_general(onehot, hlast, (((0,), (0,)), ((), ())),
                           preferred_element_type=jnp.float32)
    counts = jnp.sum(onehot, axis=0, keepdims=True)
    pooled = sums / jnp.maximum(counts, 1.0).T
    o_ref[...] = (jnp.dot(pooled, lw_ref[...],
                          preferred_element_type=jnp.float32) + lb_ref[...])


@functools.cache
def _tc_final(num_graphs, num_classes):
    return pl.pallas_call(
        functools.partial(_tc_final_body, num_graphs=num_graphs),
        out_shape=jax.ShapeDtypeStruct((num_graphs, num_classes), jnp.float32),
    )


def kernel(x, edge_index, batch, W_rel, b_rel, W_root, gamma, beta, lin_W, lin_b):
    n, d = x.shape
    e = edge_index.shape[1]
    num_layers = W_rel.shape[0]
    num_graphs = 64
    num_classes = lin_W.shape[1]

    src = edge_index[0]
    dst = edge_index[1]
    zeros = jnp.zeros((n, d), jnp.float32)
    batch2d = batch.reshape(n, 1)
    agg = _sc_aggregate(n, d, e)
    layer = _tc_layer(n, d)
    final = _tc_final(num_graphs, num_classes)

    h = x
    for i in range(num_layers):
        partials = agg(h, src, dst, zeros)
        args = (partials, h, W_rel[i], b_rel[i].reshape(1, d), W_root[i],
                gamma[i].reshape(1, d), beta[i].reshape(1, d))
        if i < num_layers - 1:
            h = layer(*args)
        else:
            out = final(*args, batch2d, lin_W, lin_b.reshape(1, num_classes))
    return out


# speedup vs baseline: 8.4928x; 1.8768x over previous
import functools

import jax
import jax.numpy as jnp
from jax import lax
from jax.experimental import pallas as pl
from jax.experimental.pallas import tpu as pltpu
from jax.experimental.pallas import tpu_sc as plsc

_NC = 2
_NS = 16
_K = 80


@functools.cache
def _sc_aggregate(n, d, e):
    nw = _NC * _NS
    ew = e // nw
    steps = ew // _K
    odd = steps % 2 == 1
    groups = (steps - 1) // 2 if odd else steps // 2
    rows_main = (n // _NS) & ~7
    tail_off = rows_main * _NS
    tail_len = n - tail_off
    mesh = plsc.VectorSubcoreMesh(core_axis_name="c", subcore_axis_name="s")

    @functools.partial(
        pl.kernel,
        mesh=mesh,
        out_type=jax.ShapeDtypeStruct((_NC, n, d), jnp.float32),
        scratch_types=[
            pltpu.VMEM_SHARED((n, d), jnp.float32),
            pltpu.VMEM((ew,), jnp.int32),
            pltpu.VMEM((steps, _K), jnp.int32),
            pltpu.VMEM((_K, d), jnp.float32),
            pltpu.VMEM((_K, d), jnp.float32),
            pltpu.SemaphoreType.DMA,
            pltpu.SemaphoreType.DMA,
            pltpu.SemaphoreType.DMA,
            pltpu.SemaphoreType.DMA,
        ],
    )
    def agg(h_hbm, src_hbm, dst_hbm, zeros_hbm, out_hbm,
            aggr_sm, srcb, dstb, r0, r1, sg0, sg1, ss0, ss1):
        c = lax.axis_index("c")
        s = lax.axis_index("s")
        w = c * _NS + s
        pltpu.sync_copy(
            zeros_hbm.at[pl.ds(s * rows_main, rows_main)],
            aggr_sm.at[pl.ds(s * rows_main, rows_main)],
        )
        if tail_len:
            @pl.when(s == _NS - 1)
            def _():
                pltpu.sync_copy(
                    zeros_hbm.at[pl.ds(tail_off, tail_len)],
                    aggr_sm.at[pl.ds(tail_off, tail_len)],
                )
        plsc.subcore_barrier()
        pltpu.sync_copy(src_hbm.at[w], srcb)
        pltpu.sync_copy(dst_hbm.at[w], dstb)

        def gather_start(j, r, sem):
            pltpu.async_copy(h_hbm.at[srcb.at[pl.ds(j * _K, _K)]], r, sem)

        def gather_wait(j, r, sem):
            pltpu.make_async_copy(
                h_hbm.at[srcb.at[pl.ds(j * _K, _K)]], r, sem).wait()

        def scatter_start(j, r, sem):
            pltpu.async_copy(r, aggr_sm.at[dstb.at[j]], sem, add=True)

        def scatter_wait(j, r, sem):
            pltpu.make_async_copy(r, aggr_sm.at[dstb.at[j]], sem).wait()

        gather_start(0, r0, sg0)

        def body(g, carry):
            j = 2 * g

            @pl.when(g > 0)
            def _():
                scatter_wait(j - 1, r1, ss1)
            gather_start(j + 1, r1, sg1)
            gather_wait(j, r0, sg0)
            scatter_start(j, r0, ss0)
            gather_wait(j + 1, r1, sg1)
            scatter_start(j + 1, r1, ss1)
            scatter_wait(j, r0, ss0)

            @pl.when(g < groups - 1)
            def _():
                gather_start(j + 2, r0, sg0)
            return carry

        lax.fori_loop(0, groups, body, 0)
        last = steps - 1
        if odd:
            gather_start(last, r0, sg0)
            scatter_wait(last - 1, r1, ss1)
            gather_wait(last, r0, sg0)
            scatter_start(last, r0, ss0)
            scatter_wait(last, r0, ss0)
        else:
            scatter_wait(last, r1, ss1)

        plsc.subcore_barrier()
        pltpu.sync_copy(
            aggr_sm.at[pl.ds(s * rows_main, rows_main)],
            out_hbm.at[c].at[pl.ds(s * rows_main, rows_main)],
        )
        if tail_len:
            @pl.when(s == _NS - 1)
            def _():
                pltpu.sync_copy(
                    aggr_sm.at[pl.ds(tail_off, tail_len)],
                    out_hbm.at[c].at[pl.ds(tail_off, tail_len)],
                )

    return agg


def _tc_layer_body(p_ref, h_ref, wr_ref, br_ref, wt_ref, g_ref, b_ref, o_ref):
    aggr = p_ref[0] + p_ref[1]
    h = h_ref[...]
    y = (jnp.dot(aggr, wr_ref[...], preferred_element_type=jnp.float32)
         + jnp.dot(h, wt_ref[...], preferred_element_type=jnp.float32)
         + br_ref[...])
    m = jnp.mean(y, axis=0, keepdims=True)
    v = jnp.mean((y - m) ** 2, axis=0, keepdims=True)
    o_ref[...] = jnp.maximum(
        g_ref[...] * (y - m) * lax.rsqrt(v + 1e-5) + b_ref[...], 0.0)


@functools.cache
def _tc_layer(n, d):
    return pl.pallas_call(
        _tc_layer_body,
        out_shape=jax.ShapeDtypeStruct((n, d), jnp.float32),
    )


def _tc_final_body(p_ref, h_ref, wr_ref, br_ref, wt_ref, g_ref, b_ref,
                   batch_ref, lw_ref, lb_ref, o_ref, *, num_graphs):
    aggr = p_ref[0] + p_ref[1]
    h = h_ref[...]
    y = (jnp.dot(aggr, wr_ref[...], preferred_element_type=jnp.float32)
         + jnp.dot(h, wt_ref[...], preferred_element_type=jnp.float32)
         + br_ref[...])
    m = jnp.mean(y, axis=0, keepdims=True)
    v = jnp.mean((y - m) ** 2, axis=0, keepdims=True)
    hlast = jnp.maximum(
        g_ref[...] * (y - m) * lax.rsqrt(v + 1e-5) + b_ref[...], 0.0)
    n = hlast.shape[0]
    seg = lax.broadcasted_iota(jnp.int32, (n, num_graphs), 1)
    onehot = (batch_ref[...] == seg).astype(jnp.float32)
    sums = lax.dot_general(onehot, hlast, (((0,), (0,)), ((), ())),
                           preferred_element_type=jnp.float32)
    counts = jnp.sum(onehot, axis=0, keepdims=True)
    pooled = sums / jnp.maximum(counts, 1.0).T
    o_ref[...] = (jnp.dot(pooled, lw_ref[...],
                          preferred_element_type=jnp.float32) + lb_ref[...])


@functools.cache
def _tc_final(num_graphs, num_classes):
    return pl.pallas_call(
        functools.partial(_tc_final_body, num_graphs=num_graphs),
        out_shape=jax.ShapeDtypeStruct((num_graphs, num_classes), jnp.float32),
    )


def kernel(x, edge_index, batch, W_rel, b_rel, W_root, gamma, beta, lin_W, lin_b):
    n, d = x.shape
    e = edge_index.shape[1]
    num_layers = W_rel.shape[0]
    num_graphs = 64
    num_classes = lin_W.shape[1]

    nw = _NC * _NS
    steps = e // (nw * _K)
    src = edge_index[0].reshape(nw, e // nw)
    dst = edge_index[1].reshape(nw, steps, _K)
    zeros = jnp.zeros((n, d), jnp.float32)
    batch2d = batch.reshape(n, 1)
    agg = _sc_aggregate(n, d, e)
    layer = _tc_layer(n, d)
    final = _tc_final(num_graphs, num_classes)

    h = x
    for i in range(num_layers):
        partials = agg(h, src, dst, zeros)
        args = (partials, h, W_rel[i], b_rel[i].reshape(1, d), W_root[i],
                gamma[i].reshape(1, d), beta[i].reshape(1, d))
        if i < num_layers - 1:
            h = layer(*args)
        else:
            out = final(*args, batch2d, lin_W, lin_b.reshape(1, num_classes))
    return out
